# baseline (speedup 1.0000x reference)
import functools

import jax
import jax.numpy as jnp
from jax import lax
from jax.experimental import pallas as pl
from jax.experimental.pallas import tpu as pltpu
from jax.experimental.pallas import tpu_sc as plsc

_B = 16384
_D = 16
_H = 128
_K = 8

_NC, _NS = 2, 16
_NW = _NC * _NS
_BPW = _B // _NW

@functools.cache
def _get_sc_gather():
    mesh = plsc.VectorSubcoreMesh(core_axis_name="c", subcore_axis_name="s",
                                  num_cores=_NC, num_subcores=_NS)

    @functools.partial(
        pl.kernel,
        mesh=mesh,
        out_type=(
            jax.ShapeDtypeStruct((_B, _D), jnp.float32),
            jax.ShapeDtypeStruct((_B, _D), jnp.float32),
            jax.ShapeDtypeStruct((_B,), jnp.float32),
            jax.ShapeDtypeStruct((_B,), jnp.float32),
        ),
        scratch_types=[
            pltpu.VMEM((_BPW,), jnp.int32),
            pltpu.VMEM((_BPW,), jnp.int32),
            pltpu.VMEM((_BPW, _D), jnp.float32),
            pltpu.VMEM((_BPW, _D), jnp.float32),
            pltpu.VMEM((_BPW,), jnp.float32),
            pltpu.VMEM((_BPW,), jnp.float32),
            pltpu.SemaphoreType.DMA,
            pltpu.SemaphoreType.DMA,
            pltpu.SemaphoreType.DMA,
            pltpu.SemaphoreType.DMA,
        ],
        compiler_params=pltpu.CompilerParams(use_tc_tiling_on_sc=False),
    )
    def sc_gather(uemb, iemb, ubias, ibias, uidx, iidx,
                  ue_out, ie_out, ub_out, ib_out,
                  uidx_v, iidx_v, urows, irows, ubr, ibr,
                  s0, s1, s2, s3):
        wid = lax.axis_index("s") * _NC + lax.axis_index("c")
        base = wid * _BPW
        pltpu.sync_copy(uidx.at[pl.ds(base, _BPW)], uidx_v)
        pltpu.sync_copy(iidx.at[pl.ds(base, _BPW)], iidx_v)
        cu = pltpu.async_copy(uemb.at[uidx_v], urows, s0)
        ci = pltpu.async_copy(iemb.at[iidx_v], irows, s1)
        cub = pltpu.async_copy(ubias.at[uidx_v], ubr, s2)
        cib = pltpu.async_copy(ibias.at[iidx_v], ibr, s3)
        cu.wait()
        pltpu.sync_copy(urows, ue_out.at[pl.ds(base, _BPW)])
        ci.wait()
        pltpu.sync_copy(irows, ie_out.at[pl.ds(base, _BPW)])
        cub.wait()
        pltpu.sync_copy(ubr, ub_out.at[pl.ds(base, _BPW)])
        cib.wait()
        pltpu.sync_copy(ibr, ib_out.at[pl.ds(base, _BPW)])

    return sc_gather


_R = 2048


def _mlp_body(ue, ie, cat, ub, ib, w1, b1, w2, b2, w3, b3, w4, b4, w5, c0,
              out):
    ie_ = ie[...]
    h = jnp.maximum(jnp.dot(ie_, w1[...], preferred_element_type=jnp.float32)
                    + b1[...], 0.0)
    h = jnp.maximum(jnp.dot(h, w2[...], preferred_element_type=jnp.float32)
                    + b2[...], 0.0)
    ws = jnp.dot(h, w3[...], preferred_element_type=jnp.float32) + b3[...]
    z = jnp.concatenate([ue[...] * ie_, ws * cat[...]], axis=1)
    h4 = jnp.maximum(jnp.dot(z, w4[...], preferred_element_type=jnp.float32)
                     + b4[...], 0.0)
    mlp = jnp.sum(h4 * w5[...], axis=1)
    score = mlp + ub[...] + ib[...] + c0[0, 0]
    out[...] = jax.nn.sigmoid(score)


_mlp = pl.pallas_call(
    _mlp_body,
    grid=(_B // _R,),
    in_specs=[
        pl.BlockSpec((_R, _D), lambda i: (i, 0)),
        pl.BlockSpec((_R, _D), lambda i: (i, 0)),
        pl.BlockSpec((_R, _K), lambda i: (i, 0)),
        pl.BlockSpec((_R,), lambda i: (i,)),
        pl.BlockSpec((_R,), lambda i: (i,)),
        pl.BlockSpec((_D, 2 * _H), lambda i: (0, 0)),
        pl.BlockSpec((1, 2 * _H), lambda i: (0, 0)),
        pl.BlockSpec((2 * _H, _H), lambda i: (0, 0)),
        pl.BlockSpec((1, _H), lambda i: (0, 0)),
        pl.BlockSpec((_H, _K), lambda i: (0, 0)),
        pl.BlockSpec((1, _K), lambda i: (0, 0)),
        pl.BlockSpec((_D + _K, _H), lambda i: (0, 0)),
        pl.BlockSpec((1, _H), lambda i: (0, 0)),
        pl.BlockSpec((1, _H), lambda i: (0, 0)),
        pl.BlockSpec(memory_space=pltpu.SMEM),
    ],
    out_specs=pl.BlockSpec((_R,), lambda i: (i,)),
    out_shape=jax.ShapeDtypeStruct((_B,), jnp.float32),
)


def kernel(user, item, item_cat_vector, user_emb, item_emb, user_bias,
           item_bias, global_bias, W1, b1, W2, b2, W3, b3, W4, b4, W5, b5):
    u = user.reshape(-1).astype(jnp.int32)
    it = item.reshape(-1).astype(jnp.int32)
    ue, ie, ub, ib = _get_sc_gather()(user_emb, item_emb,
                                      user_bias.reshape(-1),
                                      item_bias.reshape(-1), u, it)
    c0 = (global_bias + b5[0]).reshape(1, 1).astype(jnp.float32)
    return _mlp(ue, ie, item_cat_vector, ub, ib,
                W1, b1.reshape(1, -1), W2, b2.reshape(1, -1),
                W3[:, :_K], b3[:_K].reshape(1, -1),
                W4[:_D + _K], b4.reshape(1, -1), W5.reshape(1, -1), c0)
